# baseline (speedup 1.0000x reference)
import functools

import jax
import jax.numpy as jnp
from jax.experimental import pallas as pl
from jax.experimental.pallas import tpu as pltpu

N_NODES = 10000
N_GRAPHS = 64
EMBED = 256
N_PAD = 10240
BLK = 1024


def _gconv_dense_body(agg_ref, x_ref, wrel_ref, wroot_ref, b_ref, o_ref):
    acc = jnp.dot(agg_ref[...], wrel_ref[...], preferred_element_type=jnp.float32)
    acc += jnp.dot(x_ref[...], wroot_ref[...], preferred_element_type=jnp.float32)
    o_ref[...] = jnp.maximum(acc + b_ref[...], 0.0)


def _gconv_dense(agg, x_in, w_rel, w_root, b):
    k_in = x_in.shape[1]
    grid = N_PAD // BLK
    return pl.pallas_call(
        _gconv_dense_body,
        grid=(grid,),
        in_specs=[
            pl.BlockSpec((BLK, k_in), lambda i: (i, 0)),
            pl.BlockSpec((BLK, k_in), lambda i: (i, 0)),
            pl.BlockSpec((k_in, EMBED), lambda i: (0, 0)),
            pl.BlockSpec((k_in, EMBED), lambda i: (0, 0)),
            pl.BlockSpec((1, EMBED), lambda i: (0, 0)),
        ],
        out_specs=pl.BlockSpec((BLK, EMBED), lambda i: (i, 0)),
        out_shape=jax.ShapeDtypeStruct((N_PAD, EMBED), jnp.float32),
    )(agg, x_in, w_rel, w_root, b.reshape(1, EMBED))


def _pad_rows(a):
    return jnp.pad(a, ((0, N_PAD - N_NODES), (0, 0)))


def kernel(x, edge_index, batch, W_rel1, b1, W_root1, W_rel2, b2, W_root2,
           W_rel3, b3, W_root3, p3, lin1_w, lin1_b, lin2_w, lin2_b, lin3_w,
           lin3_b):
    src, dst = edge_index[0], edge_index[1]

    counts = jax.ops.segment_sum(jnp.ones_like(batch), batch,
                                 num_segments=N_GRAPHS)
    starts = jnp.concatenate(
        [jnp.zeros((1,), counts.dtype), jnp.cumsum(counts)[:-1]])
    k_per = (counts * 8 + 9) // 10
    pos = jnp.arange(N_NODES, dtype=counts.dtype)
    rank = pos - starts[batch]
    mask = rank < k_per[batch]

    def gmp(h):
        return jax.ops.segment_max(h, batch, num_segments=N_GRAPHS)

    def gap(h):
        s = jax.ops.segment_sum(h, batch, num_segments=N_GRAPHS)
        return s / jnp.maximum(counts.astype(jnp.float32), 1.0)[:, None]

    def agg_of(h):
        return jax.ops.segment_sum(h[src], dst, num_segments=N_NODES)

    h1 = _gconv_dense(_pad_rows(agg_of(x)), _pad_rows(x), W_rel1, W_root1,
                      b1)[:N_NODES]
    x1 = jnp.concatenate([gmp(h1), gap(h1)], axis=1)
    h2 = _gconv_dense(_pad_rows(agg_of(h1)), _pad_rows(h1), W_rel2, W_root2,
                      b2)[:N_NODES]
    x2 = jnp.concatenate([gmp(h2), gap(h2)], axis=1)
    h3 = _gconv_dense(_pad_rows(agg_of(h2)), _pad_rows(h2), W_rel3, W_root3,
                      b3)[:N_NODES]

    score = jnp.tanh(h3 @ p3 / jnp.linalg.norm(p3))
    order = jnp.lexsort((-score, batch))
    xp = h3[order] * score[order][:, None]
    bp = batch[order]
    x3_max = jax.ops.segment_max(jnp.where(mask[:, None], xp, -jnp.inf), bp,
                                 num_segments=N_GRAPHS)
    s = jax.ops.segment_sum(jnp.where(mask[:, None], xp, 0.0), bp,
                            num_segments=N_GRAPHS)
    c = jax.ops.segment_sum(jnp.where(mask, 1.0, 0.0)[:, None], bp,
                            num_segments=N_GRAPHS)
    x3 = jnp.concatenate([x3_max, s / jnp.maximum(c, 1.0)], axis=1)

    xs = x1 + x2 + x3
    h = jax.nn.relu(xs @ lin1_w + lin1_b)
    h = jax.nn.relu(h @ lin2_w + lin2_b)
    return jax.nn.sigmoid(h @ lin3_w + lin3_b)[:, 0]


# speedup vs baseline: 3.4393x; 3.4393x over previous
import functools

import jax
import jax.numpy as jnp
from jax import lax
from jax.experimental import pallas as pl
from jax.experimental.pallas import tpu as pltpu
from jax.experimental.pallas import tpu_sc as plsc

N_NODES = 10000
N_EDGES = 320000
N_GRAPHS = 64
EMBED = 256
HALF = 128
N_PAD = 10240
BLK = 1024
N_BLKS = N_PAD // BLK
EC = 128
N_CHUNKS = N_EDGES // EC
RPT = 624
TAIL0 = 16 * RPT
TAILN = N_NODES - TAIL0
BATCH_PAD = 1 << 30
NEG_INF = float("-inf")

_sc_mesh = plsc.VectorSubcoreMesh(core_axis_name="c", subcore_axis_name="s")


def _segsum23(table, src, dst, zeros):
    @functools.partial(
        pl.kernel,
        out_type=jax.ShapeDtypeStruct((2 * N_PAD, HALF), jnp.float32),
        mesh=_sc_mesh,
        scratch_types=[
            pltpu.VMEM((EC,), jnp.int32),
            pltpu.VMEM((EC,), jnp.int32),
            pltpu.VMEM((EC,), jnp.int32),
            pltpu.VMEM((EC, HALF), jnp.float32),
            pltpu.VMEM_SHARED((N_NODES, HALF), jnp.float32),
            pltpu.SemaphoreType.DMA,
        ],
    )
    def body(table_h, src_h, dst_h, zeros_h, out_h, src_v, idx_v, dst_v,
             rows_v, acc_sh, sem):
        c = lax.axis_index("c")
        s = lax.axis_index("s")
        pltpu.sync_copy(zeros_h.at[pl.ds(0, RPT)],
                        acc_sh.at[pl.ds(s * RPT, RPT)])

        @pl.when(s == 15)
        def _():
            pltpu.sync_copy(zeros_h.at[pl.ds(RPT, TAILN)],
                            acc_sh.at[pl.ds(TAIL0, TAILN)])

        plsc.subcore_barrier()

        def chunk(k, carry):
            cid = k * 16 + s

            @pl.when(cid < N_CHUNKS)
            def _():
                base = cid * EC
                pltpu.sync_copy(src_h.at[pl.ds(base, EC)], src_v)
                pltpu.sync_copy(dst_h.at[pl.ds(base, EC)], dst_v)

                def add_off(j, carry2):
                    idx_v[pl.ds(j * 16, 16)] = (
                        src_v[pl.ds(j * 16, 16)] + c * N_PAD)
                    return carry2

                lax.fori_loop(0, EC // 16, add_off, 0)
                pltpu.async_copy(table_h.at[idx_v], rows_v, sem).wait()
                pltpu.sync_copy(rows_v, acc_sh.at[dst_v], add=True)

            return carry

        lax.fori_loop(0, (N_CHUNKS + 15) // 16, chunk, 0)
        plsc.subcore_barrier()
        r0 = s * RPT
        pltpu.sync_copy(acc_sh.at[pl.ds(r0, RPT)],
                        out_h.at[pl.ds(c * N_PAD + r0, RPT)])

        @pl.when(s == 15)
        def _():
            pltpu.sync_copy(acc_sh.at[pl.ds(TAIL0, TAILN)],
                            out_h.at[pl.ds(c * N_PAD + TAIL0, TAILN)])

    return body(table, src, dst, zeros)


F1 = N_NODES * 4
W1 = 2496
W1T0 = 16 * W1
W1TN = F1 - W1T0


def _segsum1(x_flat, src, dst):
    @functools.partial(
        pl.kernel,
        out_type=jax.ShapeDtypeStruct((2 * F1,), jnp.float32),
        mesh=_sc_mesh,
        scratch_types=[
            pltpu.VMEM((EC,), jnp.int32),
            pltpu.VMEM((EC,), jnp.int32),
            pltpu.VMEM((EC,), jnp.int32),
            pltpu.VMEM((EC,), jnp.int32),
            pltpu.VMEM((EC,), jnp.float32),
            pltpu.VMEM((W1,), jnp.float32),
            pltpu.VMEM_SHARED((F1,), jnp.float32),
            pltpu.SemaphoreType.DMA,
        ],
    )
    def body(x_h, src_h, dst_h, out_h, src_v, dst_v, gi_v, si_v,
             col_v, stage_v, acc_sh, sem):
        c = lax.axis_index("c")
        s = lax.axis_index("s")

        def zinit(j, carry):
            stage_v[pl.ds(j * 16, 16)] = jnp.zeros((16,), jnp.float32)
            return carry

        lax.fori_loop(0, W1 // 16, zinit, 0)
        pltpu.sync_copy(stage_v, acc_sh.at[pl.ds(s * W1, W1)])

        @pl.when(s == 15)
        def _():
            pltpu.sync_copy(stage_v.at[pl.ds(0, W1TN)],
                            acc_sh.at[pl.ds(W1T0, W1TN)])

        plsc.subcore_barrier()
        wid = s * 2 + c

        def chunk(k, carry):
            cid = k * 32 + wid

            @pl.when(cid < N_CHUNKS)
            def _():
                base = cid * EC
                pltpu.sync_copy(src_h.at[pl.ds(base, EC)], src_v)
                pltpu.sync_copy(dst_h.at[pl.ds(base, EC)], dst_v)
                for j in range(4):
                    def mkidx(g, carry2):
                        sl = src_v[pl.ds(g * 16, 16)]
                        gi_v[pl.ds(g * 16, 16)] = sl * 4 + j
                        dl = dst_v[pl.ds(g * 16, 16)]
                        si_v[pl.ds(g * 16, 16)] = dl * 4 + j
                        return carry2

                    lax.fori_loop(0, EC // 16, mkidx, 0)
                    pltpu.async_copy(x_h.at[gi_v], col_v, sem).wait()
                    pltpu.sync_copy(col_v, acc_sh.at[si_v], add=True)

            return carry

        lax.fori_loop(0, (N_CHUNKS + 31) // 32, chunk, 0)
        plsc.subcore_barrier()
        r0 = s * W1
        pltpu.sync_copy(acc_sh.at[pl.ds(r0, W1)], stage_v)
        pltpu.sync_copy(stage_v, out_h.at[pl.ds(c * F1 + r0, W1)])

        @pl.when(s == 15)
        def _():
            pltpu.sync_copy(acc_sh.at[pl.ds(W1T0, W1TN)],
                            stage_v.at[pl.ds(0, W1TN)])
            pltpu.sync_copy(stage_v.at[pl.ds(0, W1TN)],
                            out_h.at[pl.ds(c * F1 + W1T0, W1TN)])

    return body(x_flat, src, dst)


def _relu(v):
    return jnp.maximum(v, 0.0)


def _pool_update(i, batch_blk, h, mx_acc, sm_acc):
    @pl.when(i == 0)
    def _():
        mx_acc[...] = jnp.full((N_GRAPHS, EMBED), NEG_INF, jnp.float32)
        sm_acc[...] = jnp.zeros((N_GRAPHS, EMBED), jnp.float32)

    b0 = jnp.min(batch_blk)
    b1 = jnp.minimum(jnp.max(batch_blk) + 1, N_GRAPHS)

    def gbody(g, carry):
        m = batch_blk == g
        mx = jnp.max(jnp.where(m, h, NEG_INF), axis=0, keepdims=True)
        sm = jnp.sum(jnp.where(m, h, 0.0), axis=0, keepdims=True)
        mx_acc[pl.ds(g, 1), :] = jnp.maximum(mx_acc[pl.ds(g, 1), :], mx)
        sm_acc[pl.ds(g, 1), :] = sm_acc[pl.ds(g, 1), :] + sm
        return carry

    lax.fori_loop(b0, b1, gbody, 0)


def _gconv23_body(batch_ref, agg_ref, x_ref, wrel_ref, wroot_ref, b_ref,
                  invc_ref, h_ref, pool_ref, mx_acc, sm_acc):
    i = pl.program_id(0)
    agg = jnp.concatenate([agg_ref[0], agg_ref[1]], axis=1)
    xin = jnp.concatenate([x_ref[0], x_ref[1]], axis=1)
    h = _relu(jnp.dot(agg, wrel_ref[...], preferred_element_type=jnp.float32)
              + jnp.dot(xin, wroot_ref[...],
                        preferred_element_type=jnp.float32)
              + b_ref[...])
    h_ref[0] = h[:, :HALF]
    h_ref[1] = h[:, HALF:]
    _pool_update(i, batch_ref[...], h, mx_acc, sm_acc)

    @pl.when(i == pl.num_programs(0) - 1)
    def _():
        pool_ref[0] = mx_acc[...]
        pool_ref[1] = sm_acc[...] * invc_ref[...]


def _gconv1_body(batch_ref, agg_ref, x_ref, wrel_ref, wroot_ref, b_ref,
                 invc_ref, h_ref, pool_ref, mx_acc, sm_acc):
    i = pl.program_id(0)
    agg = agg_ref[0] + agg_ref[1]
    h = _relu(jnp.dot(agg, wrel_ref[...], preferred_element_type=jnp.float32)
              + jnp.dot(x_ref[...], wroot_ref[...],
                        preferred_element_type=jnp.float32)
              + b_ref[...])
    h_ref[0] = h[:, :HALF]
    h_ref[1] = h[:, HALF:]
    _pool_update(i, batch_ref[...], h, mx_acc, sm_acc)

    @pl.when(i == pl.num_programs(0) - 1)
    def _():
        pool_ref[0] = mx_acc[...]
        pool_ref[1] = sm_acc[...] * invc_ref[...]


def _gconv3_body(agg_ref, x_ref, wrel_ref, wroot_ref, b_ref, h_ref):
    agg = jnp.concatenate([agg_ref[0], agg_ref[1]], axis=1)
    xin = jnp.concatenate([x_ref[0], x_ref[1]], axis=1)
    h = _relu(jnp.dot(agg, wrel_ref[...], preferred_element_type=jnp.float32)
              + jnp.dot(xin, wroot_ref[...],
                        preferred_element_type=jnp.float32)
              + b_ref[...])
    h_ref[0] = h[:, :HALF]
    h_ref[1] = h[:, HALF:]


def _split_spec(k):
    return pl.BlockSpec((2, BLK, k), lambda i: (0, i, 0))


def _full_spec(shape):
    nd = len(shape)
    return pl.BlockSpec(shape, lambda i: (0,) * nd)


def _gconv23(batch_col, agg_s, h_s, w_rel, w_root, b, invc, with_pool=True):
    body = _gconv23_body if with_pool else _gconv3_body
    in_specs = [
        _split_spec(HALF),
        _split_spec(HALF),
        _full_spec((EMBED, EMBED)),
        _full_spec((EMBED, EMBED)),
        _full_spec((1, EMBED)),
    ]
    out_shapes = [jax.ShapeDtypeStruct((2, N_PAD, HALF), jnp.float32)]
    out_specs = [_split_spec(HALF)]
    scratch = []
    if with_pool:
        in_specs = [pl.BlockSpec((BLK, 1), lambda i: (i, 0))] + in_specs
        in_specs.append(_full_spec((N_GRAPHS, 1)))
        out_shapes.append(
            jax.ShapeDtypeStruct((2, N_GRAPHS, EMBED), jnp.float32))
        out_specs.append(pl.BlockSpec((2, N_GRAPHS, EMBED),
                                      lambda i: (0, 0, 0)))
        scratch = [pltpu.VMEM((N_GRAPHS, EMBED), jnp.float32),
                   pltpu.VMEM((N_GRAPHS, EMBED), jnp.float32)]
    args = ([batch_col] if with_pool else []) + [
        agg_s, h_s, w_rel, w_root, b.reshape(1, EMBED)]
    if with_pool:
        args.append(invc)
    res = pl.pallas_call(
        body,
        grid=(N_BLKS,),
        in_specs=in_specs,
        out_specs=out_specs,
        out_shape=out_shapes,
        scratch_shapes=scratch,
    )(*args)
    return res if with_pool else res[0]


def _gconv1(batch_col, agg_s, x_pad, w_rel, w_root, b, invc):
    res = pl.pallas_call(
        _gconv1_body,
        grid=(N_BLKS,),
        in_specs=[
            pl.BlockSpec((BLK, 1), lambda i: (i, 0)),
            _split_spec(4),
            pl.BlockSpec((BLK, 4), lambda i: (i, 0)),
            _full_spec((4, EMBED)),
            _full_spec((4, EMBED)),
            _full_spec((1, EMBED)),
            _full_spec((N_GRAPHS, 1)),
        ],
        out_specs=[_split_spec(HALF),
                   pl.BlockSpec((2, N_GRAPHS, EMBED), lambda i: (0, 0, 0))],
        out_shape=[jax.ShapeDtypeStruct((2, N_PAD, HALF), jnp.float32),
                   jax.ShapeDtypeStruct((2, N_GRAPHS, EMBED), jnp.float32)],
        scratch_shapes=[pltpu.VMEM((N_GRAPHS, EMBED), jnp.float32),
                        pltpu.VMEM((N_GRAPHS, EMBED), jnp.float32)],
    )(batch_col, agg_s, x_pad, w_rel, w_root, b.reshape(1, EMBED), invc)
    return res


def kernel(x, edge_index, batch, W_rel1, b1, W_root1, W_rel2, b2, W_root2,
           W_rel3, b3, W_root3, p3, lin1_w, lin1_b, lin2_w, lin2_b, lin3_w,
           lin3_b):
    src, dst = edge_index[0], edge_index[1]

    starts = jnp.searchsorted(batch, jnp.arange(N_GRAPHS, dtype=jnp.int32),
                              side="left").astype(jnp.int32)
    ends = jnp.concatenate(
        [starts[1:], jnp.full((1,), N_NODES, jnp.int32)])
    counts = ends - starts
    invc = (1.0 / jnp.maximum(counts.astype(jnp.float32), 1.0)).reshape(
        N_GRAPHS, 1)
    k_per = (counts * 8 + 9) // 10
    pos = jnp.arange(N_NODES, dtype=jnp.int32)
    rank = pos - starts[batch]
    mask = rank < k_per[batch]

    batch_col = jnp.pad(batch.reshape(N_NODES, 1),
                        ((0, N_PAD - N_NODES), (0, 0)),
                        constant_values=BATCH_PAD)

    zeros_h = jnp.zeros((RPT + TAILN, HALF), jnp.float32)
    x_pad = jnp.pad(x, ((0, N_PAD - N_NODES), (0, 0)))

    agg1 = jnp.pad(
        _segsum1(x.reshape(-1), src, dst).reshape(2, N_NODES, 4),
        ((0, 0), (0, N_PAD - N_NODES), (0, 0)))
    h1_s, pool1 = _gconv1(batch_col, agg1, x_pad, W_rel1, W_root1, b1, invc)
    agg2 = _segsum23(h1_s.reshape(2 * N_PAD, HALF), src, dst,
                     zeros_h).reshape(2, N_PAD, HALF)
    h2_s, pool2 = _gconv23(batch_col, agg2, h1_s, W_rel2, W_root2, b2, invc)
    agg3 = _segsum23(h2_s.reshape(2 * N_PAD, HALF), src, dst,
                     zeros_h).reshape(2, N_PAD, HALF)
    h3_s = _gconv23(None, agg3, h2_s, W_rel3, W_root3, b3, None,
                    with_pool=False)

    x1 = jnp.concatenate([pool1[0], pool1[1]], axis=1)
    x2 = jnp.concatenate([pool2[0], pool2[1]], axis=1)

    h3 = jnp.concatenate([h3_s[0][:N_NODES], h3_s[1][:N_NODES]], axis=1)
    score = jnp.tanh(h3 @ p3 / jnp.linalg.norm(p3))
    order = jnp.lexsort((-score, batch))
    xp = h3[order] * score[order][:, None]
    bp = batch[order]
    x3_max = jax.ops.segment_max(jnp.where(mask[:, None], xp, NEG_INF), bp,
                                 num_segments=N_GRAPHS)
    s = jax.ops.segment_sum(jnp.where(mask[:, None], xp, 0.0), bp,
                            num_segments=N_GRAPHS)
    c = jax.ops.segment_sum(jnp.where(mask, 1.0, 0.0)[:, None], bp,
                            num_segments=N_GRAPHS)
    x3 = jnp.concatenate([x3_max, s / jnp.maximum(c, 1.0)], axis=1)

    xs = x1 + x2 + x3
    h = jax.nn.relu(xs @ lin1_w + lin1_b)
    h = jax.nn.relu(h @ lin2_w + lin2_b)
    return jax.nn.sigmoid(h @ lin3_w + lin3_b)[:, 0]


# speedup vs baseline: 4.3866x; 1.2754x over previous
import functools

import jax
import jax.numpy as jnp
from jax import lax
from jax.experimental import pallas as pl
from jax.experimental.pallas import tpu as pltpu
from jax.experimental.pallas import tpu_sc as plsc

N_NODES = 10000
N_EDGES = 320000
N_GRAPHS = 64
EMBED = 256
HALF = 128
N_PAD = 10240
BLK = 1024
N_BLKS = N_PAD // BLK
EC = 128
N_CHUNKS = N_EDGES // EC
RPT = 624
TAIL0 = 16 * RPT
TAILN = N_NODES - TAIL0
BATCH_PAD = 1 << 30
NEG_INF = float("-inf")

@functools.cache
def _sc_mesh():
    return plsc.VectorSubcoreMesh(core_axis_name="c", subcore_axis_name="s",
                                  num_cores=2, num_subcores=16)


EC2 = 256
NC2 = N_EDGES // EC2
NT2 = (NC2 // 16 + 2) // 2


@functools.cache
def _make_segsum23():
    @functools.partial(
        pl.kernel,
        out_type=jax.ShapeDtypeStruct((2 * N_PAD, HALF), jnp.float32),
        mesh=_sc_mesh(),
        scratch_types=[
            pltpu.VMEM((EC2,), jnp.int32),
            pltpu.VMEM((EC2,), jnp.int32),
            pltpu.VMEM((EC2,), jnp.int32),
            pltpu.VMEM((EC2,), jnp.int32),
            pltpu.VMEM((EC2,), jnp.int32),
            pltpu.VMEM((EC2,), jnp.int32),
            pltpu.VMEM((EC2, HALF), jnp.float32),
            pltpu.VMEM((EC2, HALF), jnp.float32),
            pltpu.VMEM_SHARED((N_NODES, HALF), jnp.float32),
            pltpu.SemaphoreType.DMA,
            pltpu.SemaphoreType.DMA,
            pltpu.SemaphoreType.DMA,
            pltpu.SemaphoreType.DMA,
        ],
    )
    def body(table_h, src_h, dst_h, zeros_h, out_h, srcA, gidxA, dstA,
             srcB, gidxB, dstB, bufA, bufB, acc_sh, gsA, gsB, ssA, ssB):
        c = lax.axis_index("c")
        s = lax.axis_index("s")
        pltpu.sync_copy(zeros_h.at[pl.ds(0, RPT)],
                        acc_sh.at[pl.ds(s * RPT, RPT)])

        @pl.when(s == 15)
        def _():
            pltpu.sync_copy(zeros_h.at[pl.ds(RPT, TAILN)],
                            acc_sh.at[pl.ds(TAIL0, TAILN)])

        plsc.subcore_barrier()

        def chunk(k, carry):
            cid = k * 16 + s

            @pl.when(cid < NC2)
            def _():
                base = cid * EC2
                pltpu.sync_copy(src_h.at[pl.ds(base, EC2)], srcA)
                pltpu.sync_copy(dst_h.at[pl.ds(base, EC2)], dstA)

                def add_off(g, carry2):
                    gidxA[pl.ds(g * 16, 16)] = (
                        srcA[pl.ds(g * 16, 16)] + c * N_PAD)
                    return carry2

                lax.fori_loop(0, EC2 // 16, add_off, 0)
                pltpu.async_copy(table_h.at[gidxA], bufA, gsA).wait()
                pltpu.sync_copy(bufA, acc_sh.at[dstA], add=True)

            return carry

        lax.fori_loop(0, (NC2 + 15) // 16, chunk, 0)

        plsc.subcore_barrier()
        r0 = s * RPT
        pltpu.sync_copy(acc_sh.at[pl.ds(r0, RPT)],
                        out_h.at[pl.ds(c * N_PAD + r0, RPT)])

        @pl.when(s == 15)
        def _():
            pltpu.sync_copy(acc_sh.at[pl.ds(TAIL0, TAILN)],
                            out_h.at[pl.ds(c * N_PAD + TAIL0, TAILN)])

    return body


F1 = N_NODES * 4
W1 = 2496
W1T0 = 16 * W1
W1TN = F1 - W1T0


def _segsum1(x_flat, src, dst):
    @functools.partial(
        pl.kernel,
        out_type=jax.ShapeDtypeStruct((2 * F1,), jnp.float32),
        mesh=_sc_mesh(),
        scratch_types=[
            pltpu.VMEM((EC,), jnp.int32),
            pltpu.VMEM((EC,), jnp.int32),
            [pltpu.VMEM((EC,), jnp.int32) for _ in range(4)],
            [pltpu.VMEM((EC,), jnp.int32) for _ in range(4)],
            [pltpu.VMEM((EC,), jnp.float32) for _ in range(4)],
            pltpu.VMEM((W1,), jnp.float32),
            pltpu.VMEM_SHARED((F1,), jnp.float32),
            pltpu.SemaphoreType.DMA,
            pltpu.SemaphoreType.DMA,
        ],
    )
    def body(x_h, src_h, dst_h, out_h, src_v, dst_v, gi_v, si_v,
             col_v, stage_v, acc_sh, gsem, ssem):
        c = lax.axis_index("c")
        s = lax.axis_index("s")

        def zinit(j, carry):
            stage_v[pl.ds(j * 16, 16)] = jnp.zeros((16,), jnp.float32)
            return carry

        lax.fori_loop(0, W1 // 16, zinit, 0)
        pltpu.sync_copy(stage_v, acc_sh.at[pl.ds(s * W1, W1)])

        @pl.when(s == 15)
        def _():
            pltpu.sync_copy(stage_v.at[pl.ds(0, W1TN)],
                            acc_sh.at[pl.ds(W1T0, W1TN)])

        plsc.subcore_barrier()
        wid = s * 2 + c

        def drain_scatters():
            for j in range(4):
                pltpu.make_async_copy(col_v[j], acc_sh.at[si_v[j]],
                                      ssem).wait()

        def chunk(k, carry):
            cid = k * 32 + wid

            @pl.when((k > 0) & ((k - 1) * 32 + wid < N_CHUNKS))
            def _():
                drain_scatters()

            @pl.when(cid < N_CHUNKS)
            def _():
                base = cid * EC
                pltpu.sync_copy(src_h.at[pl.ds(base, EC)], src_v)
                pltpu.sync_copy(dst_h.at[pl.ds(base, EC)], dst_v)
                for j in range(4):
                    def mkidx(g, carry2):
                        sl = src_v[pl.ds(g * 16, 16)]
                        gi_v[j][pl.ds(g * 16, 16)] = sl * 4 + j
                        dl = dst_v[pl.ds(g * 16, 16)]
                        si_v[j][pl.ds(g * 16, 16)] = dl * 4 + j
                        return carry2

                    lax.fori_loop(0, EC // 16, mkidx, 0)
                    pltpu.async_copy(x_h.at[gi_v[j]], col_v[j], gsem)
                for j in range(4):
                    pltpu.make_async_copy(x_h.at[gi_v[j]], col_v[j],
                                          gsem).wait()
                for j in range(4):
                    pltpu.async_copy(col_v[j], acc_sh.at[si_v[j]], ssem,
                                     add=True)

            return carry

        nk = (N_CHUNKS + 31) // 32
        lax.fori_loop(0, nk, chunk, 0)

        @pl.when((nk - 1) * 32 + wid < N_CHUNKS)
        def _():
            drain_scatters()
        plsc.subcore_barrier()
        r0 = s * W1
        pltpu.sync_copy(acc_sh.at[pl.ds(r0, W1)], stage_v)
        pltpu.sync_copy(stage_v, out_h.at[pl.ds(c * F1 + r0, W1)])

        @pl.when(s == 15)
        def _():
            pltpu.sync_copy(acc_sh.at[pl.ds(W1T0, W1TN)],
                            stage_v.at[pl.ds(0, W1TN)])
            pltpu.sync_copy(stage_v.at[pl.ds(0, W1TN)],
                            out_h.at[pl.ds(c * F1 + W1T0, W1TN)])

    return body(x_flat, src, dst)


def _relu(v):
    return jnp.maximum(v, 0.0)


def _pool_update(i, batch_blk, h, mx_acc, sm_acc):
    @pl.when(i == 0)
    def _():
        mx_acc[...] = jnp.full((N_GRAPHS, EMBED), NEG_INF, jnp.float32)
        sm_acc[...] = jnp.zeros((N_GRAPHS, EMBED), jnp.float32)

    b0 = jnp.min(batch_blk)
    b1 = jnp.minimum(jnp.max(batch_blk) + 1, N_GRAPHS)

    def gbody(g, carry):
        m = batch_blk == g
        mx = jnp.max(jnp.where(m, h, NEG_INF), axis=0, keepdims=True)
        sm = jnp.sum(jnp.where(m, h, 0.0), axis=0, keepdims=True)
        mx_acc[pl.ds(g, 1), :] = jnp.maximum(mx_acc[pl.ds(g, 1), :], mx)
        sm_acc[pl.ds(g, 1), :] = sm_acc[pl.ds(g, 1), :] + sm
        return carry

    lax.fori_loop(b0, b1, gbody, 0)


def _gconv23_body(batch_ref, agg_ref, x_ref, wrel_ref, wroot_ref, b_ref,
                  invc_ref, h_ref, pool_ref, mx_acc, sm_acc):
    i = pl.program_id(0)
    agg = jnp.concatenate([agg_ref[0], agg_ref[1]], axis=1)
    xin = jnp.concatenate([x_ref[0], x_ref[1]], axis=1)
    h = _relu(jnp.dot(agg, wrel_ref[...], preferred_element_type=jnp.float32)
              + jnp.dot(xin, wroot_ref[...],
                        preferred_element_type=jnp.float32)
              + b_ref[...])
    h_ref[0] = h[:, :HALF]
    h_ref[1] = h[:, HALF:]
    _pool_update(i, batch_ref[...], h, mx_acc, sm_acc)

    @pl.when(i == pl.num_programs(0) - 1)
    def _():
        pool_ref[0] = mx_acc[...]
        pool_ref[1] = sm_acc[...] * invc_ref[...]


def _gconv1_body(batch_ref, agg_ref, x_ref, wrel_ref, wroot_ref, b_ref,
                 invc_ref, h_ref, pool_ref, mx_acc, sm_acc):
    i = pl.program_id(0)
    agg = agg_ref[0] + agg_ref[1]
    h = _relu(jnp.dot(agg, wrel_ref[...], preferred_element_type=jnp.float32)
              + jnp.dot(x_ref[...], wroot_ref[...],
                        preferred_element_type=jnp.float32)
              + b_ref[...])
    h_ref[0] = h[:, :HALF]
    h_ref[1] = h[:, HALF:]
    _pool_update(i, batch_ref[...], h, mx_acc, sm_acc)

    @pl.when(i == pl.num_programs(0) - 1)
    def _():
        pool_ref[0] = mx_acc[...]
        pool_ref[1] = sm_acc[...] * invc_ref[...]


def _gconv3_body(agg_ref, x_ref, wrel_ref, wroot_ref, b_ref, h_ref):
    agg = jnp.concatenate([agg_ref[0], agg_ref[1]], axis=1)
    xin = jnp.concatenate([x_ref[0], x_ref[1]], axis=1)
    h = _relu(jnp.dot(agg, wrel_ref[...], preferred_element_type=jnp.float32)
              + jnp.dot(xin, wroot_ref[...],
                        preferred_element_type=jnp.float32)
              + b_ref[...])
    h_ref[0] = h[:, :HALF]
    h_ref[1] = h[:, HALF:]


def _split_spec(k):
    return pl.BlockSpec((2, BLK, k), lambda i: (0, i, 0))


def _full_spec(shape):
    nd = len(shape)
    return pl.BlockSpec(shape, lambda i: (0,) * nd)


def _gconv23(batch_col, agg_s, h_s, w_rel, w_root, b, invc, with_pool=True):
    body = _gconv23_body if with_pool else _gconv3_body
    in_specs = [
        _split_spec(HALF),
        _split_spec(HALF),
        _full_spec((EMBED, EMBED)),
        _full_spec((EMBED, EMBED)),
        _full_spec((1, EMBED)),
    ]
    out_shapes = [jax.ShapeDtypeStruct((2, N_PAD, HALF), jnp.float32)]
    out_specs = [_split_spec(HALF)]
    scratch = []
    if with_pool:
        in_specs = [pl.BlockSpec((BLK, 1), lambda i: (i, 0))] + in_specs
        in_specs.append(_full_spec((N_GRAPHS, 1)))
        out_shapes.append(
            jax.ShapeDtypeStruct((2, N_GRAPHS, EMBED), jnp.float32))
        out_specs.append(pl.BlockSpec((2, N_GRAPHS, EMBED),
                                      lambda i: (0, 0, 0)))
        scratch = [pltpu.VMEM((N_GRAPHS, EMBED), jnp.float32),
                   pltpu.VMEM((N_GRAPHS, EMBED), jnp.float32)]
    args = ([batch_col] if with_pool else []) + [
        agg_s, h_s, w_rel, w_root, b.reshape(1, EMBED)]
    if with_pool:
        args.append(invc)
    res = pl.pallas_call(
        body,
        grid=(N_BLKS,),
        in_specs=in_specs,
        out_specs=out_specs,
        out_shape=out_shapes,
        scratch_shapes=scratch,
    )(*args)
    return res if with_pool else res[0]


def _gconv1(batch_col, agg_s, x_pad, w_rel, w_root, b, invc):
    res = pl.pallas_call(
        _gconv1_body,
        grid=(N_BLKS,),
        in_specs=[
            pl.BlockSpec((BLK, 1), lambda i: (i, 0)),
            _split_spec(4),
            pl.BlockSpec((BLK, 4), lambda i: (i, 0)),
            _full_spec((4, EMBED)),
            _full_spec((4, EMBED)),
            _full_spec((1, EMBED)),
            _full_spec((N_GRAPHS, 1)),
        ],
        out_specs=[_split_spec(HALF),
                   pl.BlockSpec((2, N_GRAPHS, EMBED), lambda i: (0, 0, 0))],
        out_shape=[jax.ShapeDtypeStruct((2, N_PAD, HALF), jnp.float32),
                   jax.ShapeDtypeStruct((2, N_GRAPHS, EMBED), jnp.float32)],
        scratch_shapes=[pltpu.VMEM((N_GRAPHS, EMBED), jnp.float32),
                        pltpu.VMEM((N_GRAPHS, EMBED), jnp.float32)],
    )(batch_col, agg_s, x_pad, w_rel, w_root, b.reshape(1, EMBED), invc)
    return res


def _final_body(h3_ref, batch_ref, p3_ref, starts_ref, kper_ref, invc3_ref,
                x12_ref, w1_ref, b1_ref, w2_ref, b2_ref, w3_ref, b3_ref,
                out_ref, score_s, sel_s, mx_acc):
    f32, i32, u32 = jnp.float32, jnp.int32, jnp.uint32
    p3 = p3_ref[...]
    nrm = jnp.sqrt(jnp.sum(p3 * p3))
    raw = (jnp.dot(h3_ref[0], p3[:HALF], preferred_element_type=f32)
           + jnp.dot(h3_ref[1], p3[HALF:], preferred_element_type=f32))
    score = jnp.tanh(raw / nrm)
    score_s[...] = score

    u = lax.bitcast_convert_type(score, u32)
    ukey = jnp.where((u >> 31) > 0, ~u, u | jnp.uint32(0x80000000))

    bcol = batch_ref[...]
    gids = lax.broadcasted_iota(i32, (1, N_GRAPHS), 1)
    ohf = jnp.where(bcol == gids, 1.0, 0.0).astype(f32)
    kperc = kper_ref[...]

    def tnode_of(tt):
        vhi = (tt >> 16).astype(f32)
        vlo = (tt & jnp.uint32(0xFFFF)).astype(f32)
        thi = jnp.dot(ohf, vhi, preferred_element_type=f32)
        tlo = jnp.dot(ohf, vlo, preferred_element_type=f32)
        return (thi.astype(u32) << 16) | tlo.astype(u32)

    def cnt_of(indf):
        return lax.dot_general(ohf, indf, (((0,), (0,)), ((), ())),
                               preferred_element_type=f32)

    def bit_body(t, v):
        bit = (31 - t).astype(u32)
        tt = v | (jnp.uint32(1) << bit)
        tnode = tnode_of(tt)
        indf = jnp.where(ukey >= tnode, 1.0, 0.0)
        return jnp.where(cnt_of(indf) >= kperc, tt, v)

    v = lax.fori_loop(0, 32, bit_body,
                      jnp.zeros((N_GRAPHS, 1), u32))

    tnode = tnode_of(v)
    gt = ukey > tnode
    eq = ukey == tnode
    eqf = jnp.where(eq, 1.0, 0.0)
    n_gt = cnt_of(jnp.where(gt, 1.0, 0.0))
    tie_g = kperc - n_gt

    e_incl = eqf
    shift = 1
    while shift < N_PAD:
        e_incl = e_incl + jnp.concatenate(
            [jnp.zeros((shift, 1), jnp.float32), e_incl[:N_PAD - shift]],
            axis=0)
        shift *= 2
    e_excl = e_incl - eqf

    icol = lax.broadcasted_iota(i32, (N_PAD, 1), 0)
    ltf = jnp.where(icol < starts_ref[...], 1.0, 0.0)
    base_g = lax.dot_general(ltf, eqf, (((0,), (0,)), ((), ())),
                             preferred_element_type=f32)
    base_node = jnp.dot(ohf, base_g, preferred_element_type=f32)
    tie_node = jnp.dot(ohf, tie_g, preferred_element_type=f32)
    rank = e_excl - base_node

    selb = (gt | (eq & (rank < tie_node))) & (bcol < N_GRAPHS)
    sel_s[...] = jnp.where(selb, 1.0, 0.0)
    selw = jnp.where(selb, score, 0.0)

    wsel = ohf * selw
    s0 = lax.dot_general(wsel, h3_ref[0], (((0,), (0,)), ((), ())),
                         preferred_element_type=f32)
    s1 = lax.dot_general(wsel, h3_ref[1], (((0,), (0,)), ((), ())),
                         preferred_element_type=f32)
    x3_mean = jnp.concatenate([s0, s1], axis=1) * invc3_ref[...]

    mx_acc[...] = jnp.full((N_GRAPHS, EMBED), NEG_INF, jnp.float32)

    def blk(b, carry):
        r0 = pl.multiple_of(b * BLK, BLK)
        h0b = h3_ref[0, pl.ds(r0, BLK), :]
        h1b = h3_ref[1, pl.ds(r0, BLK), :]
        scb = score_s[pl.ds(r0, BLK), :]
        seb = sel_s[pl.ds(r0, BLK), :]
        bb = batch_ref[pl.ds(r0, BLK), :]
        xp0 = h0b * scb
        xp1 = h1b * scb
        g0 = jnp.min(bb)
        g1 = jnp.minimum(jnp.max(bb) + 1, N_GRAPHS)

        def gbody(g, carry2):
            m = (bb == g) & (seb > 0.5)
            m0 = jnp.max(jnp.where(m, xp0, NEG_INF), axis=0, keepdims=True)
            m1 = jnp.max(jnp.where(m, xp1, NEG_INF), axis=0, keepdims=True)
            mq = jnp.concatenate([m0, m1], axis=1)
            mx_acc[pl.ds(g, 1), :] = jnp.maximum(mx_acc[pl.ds(g, 1), :], mq)
            return carry2

        lax.fori_loop(g0, g1, gbody, 0)
        return carry

    lax.fori_loop(0, N_BLKS, blk, 0)

    xs = x12_ref[...] + jnp.concatenate([mx_acc[...], x3_mean], axis=1)
    h = _relu(jnp.dot(xs, w1_ref[...], preferred_element_type=f32)
              + b1_ref[...])
    h = _relu(jnp.dot(h, w2_ref[...], preferred_element_type=f32)
              + b2_ref[...])
    out_ref[...] = jax.nn.sigmoid(
        jnp.dot(h, w3_ref[...], preferred_element_type=f32) + b3_ref[...])


def _final(h3_s, batch_col, p3c, starts_row, kper_col, invc3, x12,
           lin1_w, lin1_b, lin2_w, lin2_b, lin3_w, lin3_b):
    shapes = [a.shape for a in (h3_s, batch_col, p3c, starts_row, kper_col,
                                invc3, x12, lin1_w, lin1_b, lin2_w, lin2_b,
                                lin3_w, lin3_b)]
    return pl.pallas_call(
        _final_body,
        grid=(1,),
        in_specs=[pl.BlockSpec(s, lambda i, nd=len(s): (0,) * nd)
                  for s in shapes],
        out_specs=pl.BlockSpec((N_GRAPHS, 1), lambda i: (0, 0)),
        out_shape=jax.ShapeDtypeStruct((N_GRAPHS, 1), jnp.float32),
        scratch_shapes=[pltpu.VMEM((N_PAD, 1), jnp.float32),
                        pltpu.VMEM((N_PAD, 1), jnp.float32),
                        pltpu.VMEM((N_GRAPHS, EMBED), jnp.float32)],
    )(h3_s, batch_col, p3c, starts_row, kper_col, invc3, x12, lin1_w,
      lin1_b, lin2_w, lin2_b, lin3_w, lin3_b)


def kernel(x, edge_index, batch, W_rel1, b1, W_root1, W_rel2, b2, W_root2,
           W_rel3, b3, W_root3, p3, lin1_w, lin1_b, lin2_w, lin2_b, lin3_w,
           lin3_b):
    src, dst = edge_index[0], edge_index[1]

    starts = jnp.searchsorted(batch, jnp.arange(N_GRAPHS, dtype=jnp.int32),
                              side="left").astype(jnp.int32)
    ends = jnp.concatenate(
        [starts[1:], jnp.full((1,), N_NODES, jnp.int32)])
    counts = ends - starts
    invc = (1.0 / jnp.maximum(counts.astype(jnp.float32), 1.0)).reshape(
        N_GRAPHS, 1)
    k_per = (counts * 8 + 9) // 10

    batch_col = jnp.pad(batch.reshape(N_NODES, 1),
                        ((0, N_PAD - N_NODES), (0, 0)),
                        constant_values=BATCH_PAD)

    zeros_h = jnp.zeros((RPT + TAILN, HALF), jnp.float32)
    x_pad = jnp.pad(x, ((0, N_PAD - N_NODES), (0, 0)))

    agg1 = jnp.pad(
        _segsum1(x.reshape(-1), src, dst).reshape(2, N_NODES, 4),
        ((0, 0), (0, N_PAD - N_NODES), (0, 0)))
    h1_s, pool1 = _gconv1(batch_col, agg1, x_pad, W_rel1, W_root1, b1, invc)
    agg2 = _make_segsum23()(h1_s.reshape(2 * N_PAD, HALF), src, dst,
                     zeros_h).reshape(2, N_PAD, HALF)
    h2_s, pool2 = _gconv23(batch_col, agg2, h1_s, W_rel2, W_root2, b2, invc)
    agg3 = _make_segsum23()(h2_s.reshape(2 * N_PAD, HALF), src, dst,
                     zeros_h).reshape(2, N_PAD, HALF)
    h3_s = _gconv23(None, agg3, h2_s, W_rel3, W_root3, b3, None,
                    with_pool=False)

    x1 = jnp.concatenate([pool1[0], pool1[1]], axis=1)
    x2 = jnp.concatenate([pool2[0], pool2[1]], axis=1)
    pos = jnp.arange(N_NODES, dtype=jnp.int32)
    rank = pos - starts[batch]
    mask = rank < k_per[batch]
    h3 = jnp.concatenate([h3_s[0][:N_NODES], h3_s[1][:N_NODES]], axis=1)
    score = jnp.tanh(h3 @ p3 / jnp.linalg.norm(p3))
    order = jnp.lexsort((-score, batch))
    xp = h3[order] * score[order][:, None]
    bp = batch[order]
    x3_max = jax.ops.segment_max(jnp.where(mask[:, None], xp, NEG_INF), bp,
                                 num_segments=N_GRAPHS)
    s = jax.ops.segment_sum(jnp.where(mask[:, None], xp, 0.0), bp,
                            num_segments=N_GRAPHS)
    c = jax.ops.segment_sum(jnp.where(mask, 1.0, 0.0)[:, None], bp,
                            num_segments=N_GRAPHS)
    x3 = jnp.concatenate([x3_max, s / jnp.maximum(c, 1.0)], axis=1)
    xs = x1 + x2 + x3
    h = jax.nn.relu(xs @ lin1_w + lin1_b)
    h = jax.nn.relu(h @ lin2_w + lin2_b)
    return jax.nn.sigmoid(h @ lin3_w + lin3_b)[:, 0]
